# baseline (speedup 1.0000x reference)
import jax
import jax.numpy as jnp
from jax import lax
from jax.experimental import pallas as pl
from jax.experimental.pallas import tpu as pltpu
from jax.experimental.pallas import tpu_sc as plsc

N = 10000
E = 320000
D = 128
NP = 10240
NC, NS = 2, 16
NW = NC * NS
EPT = E // NW
K = 16
NCHUNK = EPT // K
RPT = NP // NS
SPT = NP // NS

_mesh = plsc.VectorSubcoreMesh(core_axis_name="c", subcore_axis_name="s")
_sc_params = pltpu.CompilerParams(needs_layout_passes=False)


def _worker_id():
    return lax.axis_index("c") * NS + lax.axis_index("s")


def _zero_fill(buf, nvec):
    zeros = jnp.zeros((16,), jnp.float32)

    def body(i, _):
        buf[pl.ds(i * 16, 16)] = zeros
        return 0

    lax.fori_loop(0, nvec, body, 0)


def _deg_body(src_hbm, dst_hbm, w_hbm, out_hbm,
              deg_o, wdeg_o, deg_i, wdeg_i,
              srcf, dstf, wf, ones, zbuf, sem):
    c = lax.axis_index("c")
    s = lax.axis_index("s")
    wid = c * NS + s

    _zero_fill(zbuf, SPT // 16)
    row0 = s * SPT
    for tbl in (deg_o, wdeg_o, deg_i, wdeg_i):
        pltpu.sync_copy(zbuf, tbl.at[pl.ds(row0, SPT)])

    pltpu.sync_copy(src_hbm.at[pl.ds(wid * EPT, EPT)], srcf)
    pltpu.sync_copy(dst_hbm.at[pl.ds(wid * EPT, EPT)], dstf)
    pltpu.sync_copy(w_hbm.at[pl.ds(wid * EPT, EPT)], wf)
    one = jnp.ones((16,), jnp.float32)
    for g in range(K // 16 + 1):
        o = min(g * 16, K - 16)
        ones[pl.ds(o, 16)] = one

    plsc.subcore_barrier()

    def chunk(q, _):
        wq = wf.at[pl.ds(q * K, K)]
        sq = srcf.at[pl.ds(q * K, K)]
        dq = dstf.at[pl.ds(q * K, K)]
        d1 = pltpu.async_copy(ones, deg_o.at[sq], sem, add=True)
        d2 = pltpu.async_copy(wq, wdeg_o.at[sq], sem, add=True)
        d3 = pltpu.async_copy(ones, deg_i.at[dq], sem, add=True)
        d4 = pltpu.async_copy(wq, wdeg_i.at[dq], sem, add=True)
        d1.wait(); d2.wait(); d3.wait(); d4.wait()
        return 0

    lax.fori_loop(0, NCHUNK, chunk, 0)

    plsc.subcore_barrier()

    for j, tbl in enumerate((deg_o, wdeg_o, deg_i, wdeg_i)):
        pltpu.sync_copy(tbl.at[pl.ds(row0, SPT)],
                        out_hbm.at[c, j, pl.ds(row0, SPT)])


_deg_call = pl.kernel(
    _deg_body,
    out_type=jax.ShapeDtypeStruct((NC, 4, NP), jnp.float32),
    mesh=_mesh,
    scratch_types=[
        pltpu.VMEM_SHARED((NP,), jnp.float32),
        pltpu.VMEM_SHARED((NP,), jnp.float32),
        pltpu.VMEM_SHARED((NP,), jnp.float32),
        pltpu.VMEM_SHARED((NP,), jnp.float32),
        pltpu.VMEM((EPT,), jnp.int32),
        pltpu.VMEM((EPT,), jnp.int32),
        pltpu.VMEM((EPT,), jnp.float32),
        pltpu.VMEM((K,), jnp.float32),
        pltpu.VMEM((SPT,), jnp.float32),
        pltpu.SemaphoreType.DMA,
    ],
    compiler_params=_sc_params,
)


def _ab_body(deg_ref, ab_ref):
    p = deg_ref[...]
    deg_o = p[0, 0] + p[1, 0]
    wdeg_o = p[0, 1] + p[1, 1]
    deg_i = p[0, 2] + p[1, 2]
    wdeg_i = p[0, 3] + p[1, 3]
    a = (jnp.where(wdeg_o > 0, wdeg_o, 1.0) ** -0.5
         * jnp.maximum(deg_o, 1.0) ** -0.5)
    b = (jnp.where(wdeg_i > 0, wdeg_i, 1.0) ** -0.5
         * jnp.maximum(deg_i, 1.0) ** -0.5)
    ab_ref[...] = jnp.stack([a, b])


_ab_call = pl.pallas_call(
    _ab_body,
    out_shape=jax.ShapeDtypeStruct((2, NP), jnp.float32),
)


def _coef_body(src_hbm, dst_hbm, w_hbm, ab_hbm, coef_hbm,
               srcf, dstf, cf, av, bv):
    wid = _worker_id()
    base = wid * EPT
    pltpu.sync_copy(src_hbm.at[pl.ds(base, EPT)], srcf)
    pltpu.sync_copy(dst_hbm.at[pl.ds(base, EPT)], dstf)
    pltpu.sync_copy(w_hbm.at[pl.ds(base, EPT)], cf)
    pltpu.sync_copy(ab_hbm.at[0], av)
    pltpu.sync_copy(ab_hbm.at[1], bv)

    def group(g, _):
        o = pl.ds(g * 16, 16)
        coef16 = (cf[o] * plsc.load_gather(av, [srcf[o]])
                  * plsc.load_gather(bv, [dstf[o]]))
        cf[o] = coef16
        return 0

    lax.fori_loop(0, EPT // 16, group, 0)
    pltpu.sync_copy(cf, coef_hbm.at[pl.ds(base, EPT)])


_coef_call = pl.kernel(
    _coef_body,
    out_type=jax.ShapeDtypeStruct((E,), jnp.float32),
    mesh=_mesh,
    scratch_types=[
        pltpu.VMEM((EPT,), jnp.int32),
        pltpu.VMEM((EPT,), jnp.int32),
        pltpu.VMEM((EPT,), jnp.float32),
        pltpu.VMEM((NP,), jnp.float32),
        pltpu.VMEM((NP,), jnp.float32),
    ],
    compiler_params=_sc_params,
)


def _spmm_body(x_hbm, src_hbm, dst_hbm, coef_hbm, out_hbm,
               acc, srcf, dstf, cf, r0, r1, r2, m0, m1, m2,
               g0, g1, g2, s0, s1, s2):
    c = lax.axis_index("c")
    s = lax.axis_index("s")
    wid = c * NS + s
    rows = (r0, r1, r2)
    msg = (m0, m1, m2)
    gsem = (g0, g1, g2)
    ssem = (s0, s1, s2)

    zeros = jnp.zeros((16,), jnp.float32)

    def zrow(i, _):
        for r in range(D // 16):
            m0[i, pl.ds(r * 16, 16)] = zeros
        return 0

    lax.fori_loop(0, K, zrow, 0)
    for q in range(RPT // K):
        pltpu.sync_copy(m0, acc.at[pl.ds(s * RPT + q * K, K)])

    pltpu.sync_copy(src_hbm.at[pl.ds(wid * EPT, EPT)], srcf)
    pltpu.sync_copy(dst_hbm.at[pl.ds(wid * EPT, EPT)], dstf)
    pltpu.sync_copy(coef_hbm.at[pl.ds(wid * EPT, EPT)], cf)

    plsc.subcore_barrier()

    def fire_gather(q, b):
        return pltpu.async_copy(x_hbm.at[srcf.at[pl.ds(q * K, K)]],
                                rows[b], gsem[b])

    def mul_scatter(q, b):
        rb = rows[b]
        mb = msg[b]

        def scale_edge(e, ce):
            for r in range(D // 16):
                sl = pl.ds(r * 16, 16)
                mb[e, sl] = rb[e, sl] * ce

        def group(g, _):
            c16 = cf[pl.ds(q * K + g * 16, 16)]
            for j in range(16):
                scale_edge(g * 16 + j, c16[j])
            return 0

        lax.fori_loop(0, K // 16, group, 0)
        if K % 16:
            c16 = cf[pl.ds(q * K + (K - 16), 16)]
            for j in range(16 - (K % 16), 16):
                scale_edge((K - 16) + j, c16[j])
        return pltpu.async_copy(mb, acc.at[dstf.at[pl.ds(q * K, K)]],
                                ssem[b], add=True)

    def wait_scatter(b):
        pltpu.make_async_copy(msg[b], acc.at[dstf.at[pl.ds(0, K)]],
                              ssem[b]).wait()

    def wait_gather(q, b):
        pltpu.make_async_copy(x_hbm.at[srcf.at[pl.ds(0, K)]],
                              rows[b], gsem[b]).wait()

    fire_gather(0, 0)
    fire_gather(1, 1)
    for t in (2, 3):
        if t >= 3:
            wait_scatter(t % 3)
        fire_gather(t, t % 3)
        wait_gather(t - 2, (t - 2) % 3)
        mul_scatter(t - 2, (t - 2) % 3)

    def main(i, _):
        for b in range(3):
            t = (4 + b) + 3 * i
            wait_scatter((1 + b) % 3)
            fire_gather(t, (1 + b) % 3)
            wait_gather(t - 2, (2 + b) % 3)
            mul_scatter(t - 2, (2 + b) % 3)
        return 0

    lax.fori_loop(0, (NCHUNK - 4) // 3, main, 0)

    for t in (NCHUNK, NCHUNK + 1):
        wait_scatter(t % 3)
        wait_gather(t - 2, (t - 2) % 3)
        mul_scatter(t - 2, (t - 2) % 3)
    wait_scatter((NCHUNK - 1) % 3)

    plsc.subcore_barrier()

    pltpu.sync_copy(acc.at[pl.ds(s * RPT, RPT)],
                    out_hbm.at[c, pl.ds(s * RPT, RPT)])


_spmm_call = pl.kernel(
    _spmm_body,
    out_type=jax.ShapeDtypeStruct((NC, NP, D), jnp.float32),
    mesh=_mesh,
    scratch_types=[
        pltpu.VMEM_SHARED((NP, D), jnp.float32),
        pltpu.VMEM((EPT,), jnp.int32),
        pltpu.VMEM((EPT,), jnp.int32),
        pltpu.VMEM((EPT,), jnp.float32),
        pltpu.VMEM((K, D), jnp.float32),
        pltpu.VMEM((K, D), jnp.float32),
        pltpu.VMEM((K, D), jnp.float32),
        pltpu.VMEM((K, D), jnp.float32),
        pltpu.VMEM((K, D), jnp.float32),
        pltpu.VMEM((K, D), jnp.float32),
        pltpu.SemaphoreType.DMA,
        pltpu.SemaphoreType.DMA,
        pltpu.SemaphoreType.DMA,
        pltpu.SemaphoreType.DMA,
        pltpu.SemaphoreType.DMA,
        pltpu.SemaphoreType.DMA,
    ],
    compiler_params=_sc_params,
)


ROWB = 400


def _layer_body(p_ref, w_ref, b_ref, lnw_ref, lnb_ref, o_ref):
    agg = p_ref[0] + p_ref[1]
    y = jnp.dot(agg, w_ref[...], preferred_element_type=jnp.float32)
    y = jnp.maximum(y + b_ref[0], 0.0)
    mu = jnp.mean(y, axis=-1, keepdims=True)
    var = jnp.mean((y - mu) ** 2, axis=-1, keepdims=True)
    o_ref[...] = (y - mu) / jnp.sqrt(var + 1e-5) * lnw_ref[0] + lnb_ref[0]


def _layer_call(p, w, b, lnw, lnb):
    h = w.shape[1]
    return pl.pallas_call(
        _layer_body,
        grid=(N // ROWB,),
        in_specs=[
            pl.BlockSpec((NC, ROWB, D), lambda i: (0, i, 0)),
            pl.BlockSpec((D, h), lambda i: (0, 0)),
            pl.BlockSpec((1, h), lambda i: (0, 0)),
            pl.BlockSpec((1, h), lambda i: (0, 0)),
            pl.BlockSpec((1, h), lambda i: (0, 0)),
        ],
        out_specs=pl.BlockSpec((ROWB, h), lambda i: (i, 0)),
        out_shape=jax.ShapeDtypeStruct((N, h), jnp.float32),
    )(p, w, b.reshape(1, h), lnw.reshape(1, h), lnb.reshape(1, h))


def kernel(feats, edge_index, edge_weight, W1, b1, W2, b2, W3, b3,
           ln1_w, ln1_b, ln2_w, ln2_b, ln3_w, ln3_b):
    src = edge_index[0]
    dst = edge_index[1]
    deg = _deg_call(src, dst, edge_weight)
    ab = _ab_call(deg)
    coef = _coef_call(src, dst, edge_weight, ab)
    h = feats
    for W, b, lnw, lnb in ((W1, b1, ln1_w, ln1_b),
                           (W2, b2, ln2_w, ln2_b),
                           (W3, b3, ln3_w, ln3_b)):
        p = _spmm_call(h, src, dst, coef)
        h = _layer_call(p, W, b, lnw, lnb)
    return h


# speedup vs baseline: 1.4698x; 1.4698x over previous
import jax
import jax.numpy as jnp
from jax import lax
from jax.experimental import pallas as pl
from jax.experimental.pallas import tpu as pltpu
from jax.experimental.pallas import tpu_sc as plsc

N = 10000
E = 320000
D = 128
NP = 10240
NC, NS = 2, 16
NW = NC * NS
EPT = E // NW
K = 40
NCHUNK = EPT // K
RPT = NP // NS
SPT = NP // NS

_mesh = plsc.VectorSubcoreMesh(core_axis_name="c", subcore_axis_name="s")
_sc_params = pltpu.CompilerParams(needs_layout_passes=False)


def _worker_id():
    return lax.axis_index("c") * NS + lax.axis_index("s")


def _zero_fill(buf, nvec):
    zeros = jnp.zeros((16,), jnp.float32)

    def body(i, _):
        buf[pl.ds(i * 16, 16)] = zeros
        return 0

    lax.fori_loop(0, nvec, body, 0)


def _deg_body(src_hbm, dst_hbm, w_hbm, out_hbm,
              deg_o, wdeg_o, deg_i, wdeg_i,
              srcf, dstf, wf, ones, zbuf, sem):
    c = lax.axis_index("c")
    s = lax.axis_index("s")
    wid = c * NS + s

    _zero_fill(zbuf, SPT // 16)
    row0 = s * SPT
    for tbl in (deg_o, wdeg_o, deg_i, wdeg_i):
        pltpu.sync_copy(zbuf, tbl.at[pl.ds(row0, SPT)])

    pltpu.sync_copy(src_hbm.at[pl.ds(wid * EPT, EPT)], srcf)
    pltpu.sync_copy(dst_hbm.at[pl.ds(wid * EPT, EPT)], dstf)
    pltpu.sync_copy(w_hbm.at[pl.ds(wid * EPT, EPT)], wf)
    one = jnp.ones((16,), jnp.float32)
    for g in range(K // 16 + 1):
        o = min(g * 16, K - 16)
        ones[pl.ds(o, 16)] = one

    plsc.subcore_barrier()

    def chunk(q, _):
        wq = wf.at[pl.ds(q * K, K)]
        sq = srcf.at[pl.ds(q * K, K)]
        dq = dstf.at[pl.ds(q * K, K)]
        d1 = pltpu.async_copy(ones, deg_o.at[sq], sem, add=True)
        d2 = pltpu.async_copy(wq, wdeg_o.at[sq], sem, add=True)
        d3 = pltpu.async_copy(ones, deg_i.at[dq], sem, add=True)
        d4 = pltpu.async_copy(wq, wdeg_i.at[dq], sem, add=True)
        d1.wait(); d2.wait(); d3.wait(); d4.wait()
        return 0

    lax.fori_loop(0, NCHUNK, chunk, 0)

    plsc.subcore_barrier()

    for j, tbl in enumerate((deg_o, wdeg_o, deg_i, wdeg_i)):
        pltpu.sync_copy(tbl.at[pl.ds(row0, SPT)],
                        out_hbm.at[c, j, pl.ds(row0, SPT)])


_deg_call = pl.kernel(
    _deg_body,
    out_type=jax.ShapeDtypeStruct((NC, 4, NP), jnp.float32),
    mesh=_mesh,
    scratch_types=[
        pltpu.VMEM_SHARED((NP,), jnp.float32),
        pltpu.VMEM_SHARED((NP,), jnp.float32),
        pltpu.VMEM_SHARED((NP,), jnp.float32),
        pltpu.VMEM_SHARED((NP,), jnp.float32),
        pltpu.VMEM((EPT,), jnp.int32),
        pltpu.VMEM((EPT,), jnp.int32),
        pltpu.VMEM((EPT,), jnp.float32),
        pltpu.VMEM((K,), jnp.float32),
        pltpu.VMEM((SPT,), jnp.float32),
        pltpu.SemaphoreType.DMA,
    ],
    compiler_params=_sc_params,
)


def _ab_body(deg_ref, ab_ref):
    p = deg_ref[...]
    deg_o = p[0, 0] + p[1, 0]
    wdeg_o = p[0, 1] + p[1, 1]
    deg_i = p[0, 2] + p[1, 2]
    wdeg_i = p[0, 3] + p[1, 3]
    a = (jnp.where(wdeg_o > 0, wdeg_o, 1.0) ** -0.5
         * jnp.maximum(deg_o, 1.0) ** -0.5)
    b = (jnp.where(wdeg_i > 0, wdeg_i, 1.0) ** -0.5
         * jnp.maximum(deg_i, 1.0) ** -0.5)
    ab_ref[...] = jnp.stack([a, b])


_ab_call = pl.pallas_call(
    _ab_body,
    out_shape=jax.ShapeDtypeStruct((2, NP), jnp.float32),
)


def _coef_body(src_hbm, dst_hbm, w_hbm, ab_hbm, coef_hbm,
               srcf, dstf, cf, av, bv):
    wid = _worker_id()
    base = wid * EPT
    pltpu.sync_copy(src_hbm.at[pl.ds(base, EPT)], srcf)
    pltpu.sync_copy(dst_hbm.at[pl.ds(base, EPT)], dstf)
    pltpu.sync_copy(w_hbm.at[pl.ds(base, EPT)], cf)
    pltpu.sync_copy(ab_hbm.at[0], av)
    pltpu.sync_copy(ab_hbm.at[1], bv)

    def group(g, _):
        o = pl.ds(g * 16, 16)
        coef16 = (cf[o] * plsc.load_gather(av, [srcf[o]])
                  * plsc.load_gather(bv, [dstf[o]]))
        cf[o] = coef16
        return 0

    lax.fori_loop(0, EPT // 16, group, 0)
    pltpu.sync_copy(cf, coef_hbm.at[pl.ds(base, EPT)])


_coef_call = pl.kernel(
    _coef_body,
    out_type=jax.ShapeDtypeStruct((E,), jnp.float32),
    mesh=_mesh,
    scratch_types=[
        pltpu.VMEM((EPT,), jnp.int32),
        pltpu.VMEM((EPT,), jnp.int32),
        pltpu.VMEM((EPT,), jnp.float32),
        pltpu.VMEM((NP,), jnp.float32),
        pltpu.VMEM((NP,), jnp.float32),
    ],
    compiler_params=_sc_params,
)


def _spmm_body(x_hbm, src_hbm, dst_hbm, coef_hbm, out_hbm,
               acc, srcf, r0, r1, r2, m0, m1, m2,
               d0, d1, d2, c0, c1, c2,
               g0, g1, g2, s0, s1, s2):
    c = lax.axis_index("c")
    s = lax.axis_index("s")
    wid = c * NS + s
    rows = (r0, r1, r2)
    msg = (m0, m1, m2)
    dsts = (d0, d1, d2)
    cfs = (c0, c1, c2)
    gsem = (g0, g1, g2)
    ssem = (s0, s1, s2)

    zeros = jnp.zeros((16,), jnp.float32)

    def zrow(i, _):
        for r in range(D // 16):
            m0[i, pl.ds(r * 16, 16)] = zeros
        return 0

    lax.fori_loop(0, K, zrow, 0)
    for q in range(RPT // K):
        pltpu.sync_copy(m0, acc.at[pl.ds(s * RPT + q * K, K)])

    pltpu.sync_copy(src_hbm.at[pl.ds(wid * EPT, EPT)], srcf)

    plsc.subcore_barrier()

    def fire_gather(q, b):
        pltpu.async_copy(x_hbm.at[srcf.at[pl.ds(q * K, K)]],
                         rows[b], gsem[b])
        pltpu.async_copy(dst_hbm.at[pl.ds(wid * EPT + q * K, K)],
                         dsts[b], gsem[b])
        pltpu.async_copy(coef_hbm.at[pl.ds(wid * EPT + q * K, K)],
                         cfs[b], gsem[b])

    def mul_scatter(q, b):
        rb = rows[b]
        mb = msg[b]
        cb = cfs[b]

        def scale_edge(e, ce):
            for r in range(D // 16):
                sl = pl.ds(r * 16, 16)
                mb[e, sl] = rb[e, sl] * ce

        def group(g, _):
            c16 = cb[pl.ds(g * 16, 16)]
            for j in range(16):
                scale_edge(g * 16 + j, c16[j])
            return 0

        lax.fori_loop(0, K // 16, group, 0)
        if K % 16:
            c16 = cb[pl.ds(K - 16, 16)]
            for j in range(16 - (K % 16), 16):
                scale_edge((K - 16) + j, c16[j])
        return pltpu.async_copy(mb, acc.at[dsts[b].at[pl.ds(0, K)]],
                                ssem[b], add=True)

    def wait_scatter(b):
        pltpu.make_async_copy(msg[b], acc.at[dsts[b].at[pl.ds(0, K)]],
                              ssem[b]).wait()

    def wait_gather(q, b):
        pltpu.make_async_copy(x_hbm.at[srcf.at[pl.ds(0, K)]],
                              rows[b], gsem[b]).wait()
        pltpu.make_async_copy(dst_hbm.at[pl.ds(wid * EPT, K)],
                              dsts[b], gsem[b]).wait()
        pltpu.make_async_copy(coef_hbm.at[pl.ds(wid * EPT, K)],
                              cfs[b], gsem[b]).wait()

    fire_gather(0, 0)
    fire_gather(1, 1)
    for t in (2, 3):
        if t >= 3:
            wait_scatter(t % 3)
        fire_gather(t, t % 3)
        wait_gather(t - 2, (t - 2) % 3)
        mul_scatter(t - 2, (t - 2) % 3)

    def main(i, _):
        for b in range(3):
            t = (4 + b) + 3 * i
            wait_scatter((1 + b) % 3)
            fire_gather(t, (1 + b) % 3)
            wait_gather(t - 2, (2 + b) % 3)
            mul_scatter(t - 2, (2 + b) % 3)
        return 0

    lax.fori_loop(0, (NCHUNK - 4) // 3, main, 0)

    for t in (NCHUNK, NCHUNK + 1):
        wait_scatter(t % 3)
        wait_gather(t - 2, (t - 2) % 3)
        mul_scatter(t - 2, (t - 2) % 3)
    wait_scatter((NCHUNK - 1) % 3)

    plsc.subcore_barrier()

    pltpu.sync_copy(acc.at[pl.ds(s * RPT, RPT)],
                    out_hbm.at[c, pl.ds(s * RPT, RPT)])


_spmm_call = pl.kernel(
    _spmm_body,
    out_type=jax.ShapeDtypeStruct((NC, NP, D), jnp.float32),
    mesh=_mesh,
    scratch_types=[
        pltpu.VMEM_SHARED((NP, D), jnp.float32),
        pltpu.VMEM((EPT,), jnp.int32),
        pltpu.VMEM((K, D), jnp.float32),
        pltpu.VMEM((K, D), jnp.float32),
        pltpu.VMEM((K, D), jnp.float32),
        pltpu.VMEM((K, D), jnp.float32),
        pltpu.VMEM((K, D), jnp.float32),
        pltpu.VMEM((K, D), jnp.float32),
        pltpu.VMEM((K,), jnp.int32),
        pltpu.VMEM((K,), jnp.int32),
        pltpu.VMEM((K,), jnp.int32),
        pltpu.VMEM((K,), jnp.float32),
        pltpu.VMEM((K,), jnp.float32),
        pltpu.VMEM((K,), jnp.float32),
        pltpu.SemaphoreType.DMA,
        pltpu.SemaphoreType.DMA,
        pltpu.SemaphoreType.DMA,
        pltpu.SemaphoreType.DMA,
        pltpu.SemaphoreType.DMA,
        pltpu.SemaphoreType.DMA,
    ],
    compiler_params=_sc_params,
)


ROWB = 400


def _layer_body(p_ref, w_ref, b_ref, lnw_ref, lnb_ref, o_ref):
    agg = p_ref[0] + p_ref[1]
    y = jnp.dot(agg, w_ref[...], preferred_element_type=jnp.float32)
    y = jnp.maximum(y + b_ref[0], 0.0)
    mu = jnp.mean(y, axis=-1, keepdims=True)
    var = jnp.mean((y - mu) ** 2, axis=-1, keepdims=True)
    o_ref[...] = (y - mu) / jnp.sqrt(var + 1e-5) * lnw_ref[0] + lnb_ref[0]


def _layer_call(p, w, b, lnw, lnb):
    h = w.shape[1]
    return pl.pallas_call(
        _layer_body,
        grid=(N // ROWB,),
        in_specs=[
            pl.BlockSpec((NC, ROWB, D), lambda i: (0, i, 0)),
            pl.BlockSpec((D, h), lambda i: (0, 0)),
            pl.BlockSpec((1, h), lambda i: (0, 0)),
            pl.BlockSpec((1, h), lambda i: (0, 0)),
            pl.BlockSpec((1, h), lambda i: (0, 0)),
        ],
        out_specs=pl.BlockSpec((ROWB, h), lambda i: (i, 0)),
        out_shape=jax.ShapeDtypeStruct((N, h), jnp.float32),
    )(p, w, b.reshape(1, h), lnw.reshape(1, h), lnb.reshape(1, h))


def kernel(feats, edge_index, edge_weight, W1, b1, W2, b2, W3, b3,
           ln1_w, ln1_b, ln2_w, ln2_b, ln3_w, ln3_b):
    src = edge_index[0]
    dst = edge_index[1]
    deg = _deg_call(src, dst, edge_weight)
    ab = _ab_call(deg)
    coef = _coef_call(src, dst, edge_weight, ab)
    h = feats
    for W, b, lnw, lnb in ((W1, b1, ln1_w, ln1_b),
                           (W2, b2, ln2_w, ln2_b),
                           (W3, b3, ln3_w, ln3_b)):
        p = _spmm_call(h, src, dst, coef)
        h = _layer_call(p, W, b, lnw, lnb)
    return h
